# baseline (speedup 1.0000x reference)
import math

import numpy as np
import jax
import jax.numpy as jnp
from jax import lax
from jax.experimental import pallas as pl

_B, _L, _H, _D = 2, 2048, 16, 64
_FACTOR = 5
_U = min(_FACTOR * int(np.ceil(np.log(_L))), _L)
_UP = 48
_KT = 512
_NKT = _L // _KT
_NEG = np.float32(-1e30)


_CT_NP = None


def _sample_count_matrix() -> np.ndarray:
    global _CT_NP
    if _CT_NP is None:
        with jax.ensure_compile_time_eval():
            idx = np.asarray(
                jax.random.randint(jax.random.key(42), (_L, _U), 0, _L)
            ).astype(np.int64)
        c = np.zeros((_L, _L), dtype=np.int8)
        np.add.at(c, (np.repeat(np.arange(_L), _U), idx.reshape(-1)), 1)
        _CT_NP = np.ascontiguousarray(c.T)
    return _CT_NP


def _m_body(q_ref, k_ref, ct_ref, m_ref):
    q = q_ref[0, 0, :, :]
    k = k_ref[0, 0, :, :]
    m_run = jnp.full((1, _L), _NEG, jnp.float32)
    s_run = jnp.zeros((1, _L), jnp.float32)
    for t in range(_NKT):
        kt = k[t * _KT:(t + 1) * _KT, :]
        st = lax.dot_general(
            kt, q, (((1,), (1,)), ((), ())),
            precision=lax.Precision.DEFAULT,
            preferred_element_type=jnp.float32)
        cf = ct_ref[t * _KT:(t + 1) * _KT, :].astype(jnp.float32)
        m_run = jnp.maximum(
            m_run, jnp.max(jnp.where(cf > 0.0, st, _NEG), axis=0, keepdims=True))
        s_run = s_run + jnp.sum(st * cf, axis=0, keepdims=True)
    m = m_run - s_run * np.float32(1.0 / _L)
    m_ref[0, 0, :, :] = jnp.broadcast_to(m, (8, _L))


def _attn_body(m_ref, q_ref, k_ref, v_ref, o_ref):
    q = q_ref[0, 0, :, :]
    k = k_ref[0, 0, :, :]
    v = v_ref[0, 0, :, :]
    m = m_ref[0, 0, 0:1, :]

    iota = lax.broadcasted_iota(jnp.int32, (1, _L), 1)
    ranks = jnp.full((1, _L), _UP, jnp.int32)
    for i in range(_U):
        mv = jnp.max(m, axis=1, keepdims=True)
        ixv = jnp.min(jnp.where(m == mv, iota, _L), axis=1, keepdims=True)
        hit = iota == ixv
        ranks = jnp.where(hit, i, ranks)
        m = jnp.where(hit, _NEG, m)

    rr = lax.broadcasted_iota(jnp.int32, (_UP, 1), 0)
    p = (ranks == rr).astype(jnp.float32)

    qr = lax.dot_general(
        p, q, (((1,), (0,)), ((), ())),
        precision=lax.Precision.HIGHEST,
        preferred_element_type=jnp.float32)
    scores = lax.dot_general(
        qr, k, (((1,), (1,)), ((), ())),
        precision=lax.Precision.HIGHEST,
        preferred_element_type=jnp.float32)
    scores = scores * np.float32(1.0 / math.sqrt(_D))
    smax = jnp.max(scores, axis=1, keepdims=True)
    e = jnp.exp(scores - smax)
    attn = e / jnp.sum(e, axis=1, keepdims=True)
    upd = lax.dot_general(
        attn, v, (((1,), (0,)), ((), ())),
        precision=lax.Precision.HIGHEST,
        preferred_element_type=jnp.float32)

    scat = lax.dot_general(
        p, upd, (((0,), (0,)), ((), ())),
        precision=lax.Precision.HIGHEST,
        preferred_element_type=jnp.float32)
    selcol = lax.dot_general(
        p, jnp.ones((_UP, 1), jnp.float32), (((0,), (0,)), ((), ())),
        precision=lax.Precision.HIGHEST,
        preferred_element_type=jnp.float32)
    vmean = jnp.mean(v, axis=0, keepdims=True)
    o_ref[0, 0, :, :] = jnp.where(
        selcol > 0.0, scat, jnp.broadcast_to(vmean, (_L, _D)))


def _prob_attn(queries, keys, values, interpret=False):
    ct = jnp.asarray(_sample_count_matrix())
    qt = jnp.transpose(queries, (0, 2, 1, 3))
    kt = jnp.transpose(keys, (0, 2, 1, 3))
    vt = jnp.transpose(values, (0, 2, 1, 3))
    bspec = pl.BlockSpec((1, 1, _L, _D), lambda b, h: (b, h, 0, 0))
    cspec = pl.BlockSpec((_L, _L), lambda b, h: (0, 0))
    mspec = pl.BlockSpec((1, 1, 8, _L), lambda b, h: (b, h, 0, 0))
    m = pl.pallas_call(
        _m_body,
        grid=(_B, _H),
        in_specs=[bspec, bspec, cspec],
        out_specs=mspec,
        out_shape=jax.ShapeDtypeStruct((_B, _H, 8, _L), jnp.float32),
        interpret=interpret,
    )(qt, kt, ct)
    out = pl.pallas_call(
        _attn_body,
        grid=(_B, _H),
        in_specs=[mspec, bspec, bspec, bspec],
        out_specs=bspec,
        out_shape=jax.ShapeDtypeStruct((_B, _H, _L, _D), jnp.float32),
        interpret=interpret,
    )(m, qt, kt, vt)
    return jnp.transpose(out, (0, 2, 1, 3))


def kernel(queries, keys, values, attn_mask):
    return _prob_attn(queries, keys, values)


# speedup vs baseline: 1.5316x; 1.5316x over previous
import math

import numpy as np
import jax
import jax.numpy as jnp
from jax import lax
from jax.experimental import pallas as pl

_B, _L, _H, _D = 2, 2048, 16, 64
_FACTOR = 5
_U = min(_FACTOR * int(np.ceil(np.log(_L))), _L)
_UP = 48
_KT = 512
_NKT = _L // _KT
_NEG = np.float32(-1e30)


_CT_NP = None


def _sample_count_matrix() -> np.ndarray:
    global _CT_NP
    if _CT_NP is None:
        with jax.ensure_compile_time_eval():
            idx = np.asarray(
                jax.random.randint(jax.random.key(42), (_L, _U), 0, _L)
            ).astype(np.int64)
        c = np.zeros((_L, _L), dtype=np.int8)
        np.add.at(c, (np.repeat(np.arange(_L), _U), idx.reshape(-1)), 1)
        _CT_NP = np.ascontiguousarray(c.T)
    return _CT_NP


def _m_body(q_ref, k_ref, ct_ref, m_ref):
    q = q_ref[0, 0, :, :]
    k = k_ref[0, 0, :, :]
    m_run = jnp.full((1, _L), _NEG, jnp.float32)
    s_run = jnp.zeros((1, _L), jnp.float32)
    for t in range(_NKT):
        kt = k[t * _KT:(t + 1) * _KT, :]
        st = lax.dot_general(
            kt, q, (((1,), (1,)), ((), ())),
            precision=lax.Precision.DEFAULT,
            preferred_element_type=jnp.float32)
        cf = ct_ref[t * _KT:(t + 1) * _KT, :].astype(jnp.float32)
        m_run = jnp.maximum(
            m_run, jnp.max(jnp.where(cf > 0.0, st, _NEG), axis=0, keepdims=True))
        s_run = s_run + jnp.sum(st * cf, axis=0, keepdims=True)
    m = m_run - s_run * np.float32(1.0 / _L)
    m_ref[0, 0, :, :] = jnp.broadcast_to(m, (8, _L))


def _attn_body(m_ref, mt_ref, q_ref, k_ref, v_ref, o_ref):
    q = q_ref[0, 0, :, :]
    k = k_ref[0, 0, :, :]
    v = v_ref[0, 0, :, :]
    m = m_ref[0, 0, 0:1, :]

    iota = lax.broadcasted_iota(jnp.int32, (1, _L), 1)
    ranks = jnp.zeros((1, _L), jnp.float32)
    for t in range(_NKT):
        mc = mt_ref[0, 0, t * _KT:(t + 1) * _KT, 0:1]
        ic = lax.broadcasted_iota(jnp.int32, (_KT, 1), 0) + t * _KT
        beats = (mc > m) | ((mc == m) & (ic < iota))
        ranks = ranks + jnp.sum(
            jnp.where(beats, 1.0, 0.0), axis=0, keepdims=True)

    rri = lax.broadcasted_iota(jnp.int32, (_UP, 1), 0)
    rr = rri.astype(jnp.float32)
    p = (ranks == rr).astype(jnp.float32)
    p = p * jnp.where(rr < float(_U), 1.0, 0.0)

    qr = lax.dot_general(
        p, q, (((1,), (0,)), ((), ())),
        precision=lax.Precision.HIGHEST,
        preferred_element_type=jnp.float32)
    scores = lax.dot_general(
        qr, k, (((1,), (1,)), ((), ())),
        precision=lax.Precision.HIGHEST,
        preferred_element_type=jnp.float32)
    scores = scores * np.float32(1.0 / math.sqrt(_D))
    smax = jnp.max(scores, axis=1, keepdims=True)
    e = jnp.exp(scores - smax)
    attn = e / jnp.sum(e, axis=1, keepdims=True)
    upd = lax.dot_general(
        attn, v, (((1,), (0,)), ((), ())),
        precision=lax.Precision.HIGHEST,
        preferred_element_type=jnp.float32)

    scat = lax.dot_general(
        p, upd, (((0,), (0,)), ((), ())),
        precision=lax.Precision.HIGHEST,
        preferred_element_type=jnp.float32)
    selcol = lax.dot_general(
        p, jnp.ones((_UP, 1), jnp.float32), (((0,), (0,)), ((), ())),
        precision=lax.Precision.HIGHEST,
        preferred_element_type=jnp.float32)
    vmean = jnp.mean(v, axis=0, keepdims=True)
    o_ref[0, 0, :, :] = jnp.where(
        selcol > 0.0, scat, jnp.broadcast_to(vmean, (_L, _D)))


def _prob_attn(queries, keys, values, interpret=False):
    ct = jnp.asarray(_sample_count_matrix())
    qt = jnp.transpose(queries, (0, 2, 1, 3))
    kt = jnp.transpose(keys, (0, 2, 1, 3))
    vt = jnp.transpose(values, (0, 2, 1, 3))
    bspec = pl.BlockSpec((1, 1, _L, _D), lambda b, h: (b, h, 0, 0))
    cspec = pl.BlockSpec((_L, _L), lambda b, h: (0, 0))
    mspec = pl.BlockSpec((1, 1, 8, _L), lambda b, h: (b, h, 0, 0))
    m = pl.pallas_call(
        _m_body,
        grid=(_B, _H),
        in_specs=[bspec, bspec, cspec],
        out_specs=mspec,
        out_shape=jax.ShapeDtypeStruct((_B, _H, 8, _L), jnp.float32),
        interpret=interpret,
    )(qt, kt, ct)
    mt = jnp.broadcast_to(m[:, :, 0, :, None], (_B, _H, _L, 8))
    mtspec = pl.BlockSpec((1, 1, _L, 8), lambda b, h: (b, h, 0, 0))
    out = pl.pallas_call(
        _attn_body,
        grid=(_B, _H),
        in_specs=[mspec, mtspec, bspec, bspec, bspec],
        out_specs=bspec,
        out_shape=jax.ShapeDtypeStruct((_B, _H, _L, _D), jnp.float32),
        interpret=interpret,
    )(m, mt, qt, kt, vt)
    return jnp.transpose(out, (0, 2, 1, 3))


def kernel(queries, keys, values, attn_mask):
    return _prob_attn(queries, keys, values)
